# Initial kernel scaffold; baseline (speedup 1.0000x reference)
import jax
import jax.numpy as jnp
from jax.experimental import pallas as pl


def kernel(x, edge_attr, norm, W0, b0, gamma0, beta0, W1, b1, gamma1, beta1, edge_index):
    raise NotImplementedError("write your pallas kernel here")


# speedup vs baseline: 4.3810x; 4.3810x over previous
import functools

import jax
import jax.numpy as jnp
from jax import lax
from jax.experimental import pallas as pl
from jax.experimental.pallas import tpu as pltpu
from jax.experimental.pallas import tpu_sc as plsc

N = 100000
E = 3200000
D = 128
DE = 16
OUT = 128

NUM_SC = 2
NUM_SUBCORES = 16
NUM_WORKERS = NUM_SC * NUM_SUBCORES

NPAD = 100096
ACC_LEN = NPAD * DE
ZSLICE = ACC_LEN // NUM_SUBCORES

TOTAL_ELEMS = E * DE
SLAB = 8192
SLAB_ROWS = SLAB // 128
NUM_SLABS = TOTAL_ELEMS // SLAB
MAX_SLABS_PER_W = -(-NUM_SLABS // NUM_WORKERS)

EC = E // 128


def _sc_segment_sum(vals1, oidx2, zeros1):
  mesh = plsc.VectorSubcoreMesh(core_axis_name="c", subcore_axis_name="s")

  @functools.partial(
      pl.kernel,
      out_type=jax.ShapeDtypeStruct((NUM_SC * ACC_LEN,), jnp.float32),
      mesh=mesh,
      scratch_types=[
          pltpu.VMEM((SLAB,), jnp.float32),
          pltpu.VMEM((SLAB_ROWS, 128), jnp.int32),
          pltpu.VMEM_SHARED((ACC_LEN,), jnp.float32),
      ],
  )
  def sc_kernel(vals_hbm, idx_hbm, zeros_hbm, out_hbm, val_v, idx_v, acc):
    cid = lax.axis_index("c")
    sid = lax.axis_index("s")
    wid = sid * NUM_SC + cid

    z0 = sid * ZSLICE
    pltpu.sync_copy(zeros_hbm.at[pl.ds(z0, ZSLICE)], acc.at[pl.ds(z0, ZSLICE)])
    plsc.subcore_barrier()

    @pl.loop(0, MAX_SLABS_PER_W)
    def _(i):
      c = wid + i * NUM_WORKERS

      @pl.when(c < NUM_SLABS)
      def _():
        pltpu.sync_copy(vals_hbm.at[pl.ds(c * SLAB, SLAB)], val_v)
        pltpu.sync_copy(idx_hbm.at[pl.ds(c * SLAB_ROWS, SLAB_ROWS)], idx_v)
        for j in range(SLAB_ROWS):
          pltpu.sync_copy(val_v.at[pl.ds(j * 128, 128)],
                          acc.at[idx_v.at[j]], add=True)

    plsc.subcore_barrier()
    pltpu.sync_copy(acc.at[pl.ds(z0, ZSLICE)],
                    out_hbm.at[pl.ds(cid * ACC_LEN + z0, ZSLICE)])

  return sc_kernel(vals1, oidx2, zeros1)


ROW_BLOCK = 2048
PACK_ROWS = ROW_BLOCK * DE // 128


def _tc_dense_kernel(p_ref, nrep_ref, x_ref, w0h_ref, w0a_ref, b0_ref,
                     w1h_ref, w1a_ref, b1_ref, g0_ref, bt0_ref, g1_ref,
                     bt1_ref, out_ref):
  ap = (p_ref[0] + p_ref[1]) * nrep_ref[...]

  def an_dot(wa):
    parts = [
        jnp.dot(ap[:, 16 * g:16 * (g + 1)], wa,
                preferred_element_type=jnp.float32,
                precision=lax.Precision.HIGHEST)
        for g in range(8)
    ]
    t = jnp.stack(parts, axis=1)
    return t.reshape(ROW_BLOCK, OUT)

  h = x_ref[...]
  for wh, wa, b, g, bt in ((w0h_ref, w0a_ref, b0_ref, g0_ref, bt0_ref),
                           (w1h_ref, w1a_ref, b1_ref, g1_ref, bt1_ref)):
    t = (jnp.dot(h, wh[...], preferred_element_type=jnp.float32,
                 precision=lax.Precision.HIGHEST)
         + an_dot(wa[...]) + b[...])
    mean = jnp.mean(t, axis=-1, keepdims=True)
    var = jnp.mean((t - mean) ** 2, axis=-1, keepdims=True)
    t = (t - mean) / jnp.sqrt(var + 1e-5) * g[...] + bt[...]
    h = jnp.maximum(t, 0.0)
  out_ref[...] = h


def _tc_dense(partials, nrep, x, W0h, W0a, b0, W1h, W1a, b1, g0, bt0, g1,
              bt1):
  nblk = -(-N // ROW_BLOCK)
  full = lambda *shape: pl.BlockSpec(shape, lambda i: (0,) * len(shape))
  return pl.pallas_call(
      _tc_dense_kernel,
      grid=(nblk,),
      in_specs=[
          pl.BlockSpec((NUM_SC, PACK_ROWS, 128), lambda i: (0, i, 0)),
          pl.BlockSpec((PACK_ROWS, 128), lambda i: (i, 0)),
          pl.BlockSpec((ROW_BLOCK, D), lambda i: (i, 0)),
          full(D, OUT), full(DE, OUT), full(1, OUT),
          full(D, OUT), full(DE, OUT), full(1, OUT),
          full(1, OUT), full(1, OUT), full(1, OUT), full(1, OUT),
      ],
      out_specs=pl.BlockSpec((ROW_BLOCK, OUT), lambda i: (i, 0)),
      out_shape=jax.ShapeDtypeStruct((N, OUT), jnp.float32),
  )(partials, nrep, x, W0h, W0a, b0, W1h, W1a, b1, g0, bt0, g1, bt1)


@jax.jit
def kernel(x, edge_attr, norm, W0, b0, gamma0, beta0, W1, b1, gamma1, beta1,
           edge_index):
  vals1 = (edge_attr.T.reshape(2, 8, EC, 128)
           .transpose(0, 2, 1, 3).reshape(TOTAL_ELEMS))
  dst = edge_index[1].reshape(EC, 128)
  oidx = (dst[None, :, None, :] * DE
          + 8 * jnp.arange(2, dtype=jnp.int32)[:, None, None, None]
          + jnp.arange(8, dtype=jnp.int32)[None, None, :, None])
  oidx2 = oidx.reshape(TOTAL_ELEMS // 128, 128)
  zeros1 = jnp.zeros((ACC_LEN,), jnp.float32)

  partials = _sc_segment_sum(vals1, oidx2, zeros1)
  packed = partials.reshape(NUM_SC, ACC_LEN // 128, 128)

  nrep = jnp.repeat(norm.reshape(N), DE).reshape(N * DE // 128, 128)

  r2 = lambda v: v.reshape(1, OUT)
  return _tc_dense(packed, nrep, x,
                   W0[:D], W0[D:], r2(b0), W1[:D], W1[D:], r2(b1),
                   r2(gamma0), r2(beta0), r2(gamma1), r2(beta1))


# speedup vs baseline: 7.0907x; 1.6185x over previous
import functools

import jax
import jax.numpy as jnp
from jax import lax
from jax.experimental import pallas as pl
from jax.experimental.pallas import tpu as pltpu
from jax.experimental.pallas import tpu_sc as plsc

N = 100000
E = 3200000
D = 128
DE = 16
OUT = 128

NUM_SC = 2
NUM_SUBCORES = 16
NUM_WORKERS = NUM_SC * NUM_SUBCORES

NPAD = 100096
ACC_LEN = NPAD * DE
ZSLICE = ACC_LEN // NUM_SUBCORES

TOTAL_ELEMS = E * DE
SLAB = 8192
SLAB_ROWS = SLAB // 128
NUM_SLABS = TOTAL_ELEMS // SLAB
MAX_SLABS_PER_W = -(-NUM_SLABS // NUM_WORKERS)

EC = E // 128


def _sc_segment_sum(vals1, oidx2, zeros1):
  mesh = plsc.VectorSubcoreMesh(core_axis_name="c", subcore_axis_name="s")

  @functools.partial(
      pl.kernel,
      out_type=jax.ShapeDtypeStruct((NUM_SC * ACC_LEN,), jnp.float32),
      mesh=mesh,
      scratch_types=[
          pltpu.VMEM((SLAB,), jnp.float32),
          pltpu.VMEM((SLAB,), jnp.int32),
          pltpu.VMEM_SHARED((ACC_LEN,), jnp.float32),
      ],
  )
  def sc_kernel(vals_hbm, idx_hbm, zeros_hbm, out_hbm, val_v, idx_v, acc):
    cid = lax.axis_index("c")
    sid = lax.axis_index("s")
    wid = sid * NUM_SC + cid

    z0 = sid * ZSLICE
    pltpu.sync_copy(zeros_hbm.at[pl.ds(z0, ZSLICE)], acc.at[pl.ds(z0, ZSLICE)])
    plsc.subcore_barrier()

    @pl.loop(0, MAX_SLABS_PER_W)
    def _(i):
      c = wid + i * NUM_WORKERS

      @pl.when(c < NUM_SLABS)
      def _():
        pltpu.sync_copy(vals_hbm.at[pl.ds(c * SLAB, SLAB)], val_v)
        pltpu.sync_copy(idx_hbm.at[pl.ds(c * SLAB, SLAB)], idx_v)
        pltpu.sync_copy(val_v, acc.at[idx_v], add=True)

    plsc.subcore_barrier()
    pltpu.sync_copy(acc.at[pl.ds(z0, ZSLICE)],
                    out_hbm.at[pl.ds(cid * ACC_LEN + z0, ZSLICE)])

  return sc_kernel(vals1, oidx2, zeros1)


ROW_BLOCK = 2048
PACK_ROWS = ROW_BLOCK * DE // 128


def _tc_dense_kernel(p_ref, nrep_ref, x_ref, w0h_ref, w0a_ref, b0_ref,
                     w1h_ref, w1a_ref, b1_ref, g0_ref, bt0_ref, g1_ref,
                     bt1_ref, out_ref):
  ap = (p_ref[0] + p_ref[1]) * nrep_ref[...]

  def an_dot(wa):
    parts = [
        jnp.dot(ap[:, 16 * g:16 * (g + 1)], wa,
                preferred_element_type=jnp.float32,
                precision=lax.Precision.HIGHEST)
        for g in range(8)
    ]
    t = jnp.stack(parts, axis=1)
    return t.reshape(ROW_BLOCK, OUT)

  h = x_ref[...]
  for wh, wa, b, g, bt in ((w0h_ref, w0a_ref, b0_ref, g0_ref, bt0_ref),
                           (w1h_ref, w1a_ref, b1_ref, g1_ref, bt1_ref)):
    t = (jnp.dot(h, wh[...], preferred_element_type=jnp.float32,
                 precision=lax.Precision.HIGHEST)
         + an_dot(wa[...]) + b[...])
    mean = jnp.mean(t, axis=-1, keepdims=True)
    var = jnp.mean((t - mean) ** 2, axis=-1, keepdims=True)
    t = (t - mean) / jnp.sqrt(var + 1e-5) * g[...] + bt[...]
    h = jnp.maximum(t, 0.0)
  out_ref[...] = h


def _tc_dense(partials, nrep, x, W0h, W0a, b0, W1h, W1a, b1, g0, bt0, g1,
              bt1):
  nblk = -(-N // ROW_BLOCK)
  full = lambda *shape: pl.BlockSpec(shape, lambda i: (0,) * len(shape))
  return pl.pallas_call(
      _tc_dense_kernel,
      grid=(nblk,),
      in_specs=[
          pl.BlockSpec((NUM_SC, PACK_ROWS, 128), lambda i: (0, i, 0)),
          pl.BlockSpec((PACK_ROWS, 128), lambda i: (i, 0)),
          pl.BlockSpec((ROW_BLOCK, D), lambda i: (i, 0)),
          full(D, OUT), full(DE, OUT), full(1, OUT),
          full(D, OUT), full(DE, OUT), full(1, OUT),
          full(1, OUT), full(1, OUT), full(1, OUT), full(1, OUT),
      ],
      out_specs=pl.BlockSpec((ROW_BLOCK, OUT), lambda i: (i, 0)),
      out_shape=jax.ShapeDtypeStruct((N, OUT), jnp.float32),
  )(partials, nrep, x, W0h, W0a, b0, W1h, W1a, b1, g0, bt0, g1, bt1)


@jax.jit
def kernel(x, edge_attr, norm, W0, b0, gamma0, beta0, W1, b1, gamma1, beta1,
           edge_index):
  vals1 = (edge_attr.T.reshape(2, 8, EC, 128)
           .transpose(0, 2, 1, 3).reshape(TOTAL_ELEMS))
  dst = edge_index[1].reshape(EC, 128)
  oidx = (dst[None, :, None, :] * DE
          + 8 * jnp.arange(2, dtype=jnp.int32)[:, None, None, None]
          + jnp.arange(8, dtype=jnp.int32)[None, None, :, None])
  oidx2 = oidx.reshape(TOTAL_ELEMS)
  zeros1 = jnp.zeros((ACC_LEN,), jnp.float32)

  partials = _sc_segment_sum(vals1, oidx2, zeros1)
  packed = partials.reshape(NUM_SC, ACC_LEN // 128, 128)

  nrep = jnp.repeat(norm.reshape(N), DE).reshape(N * DE // 128, 128)

  r2 = lambda v: v.reshape(1, OUT)
  return _tc_dense(packed, nrep, x,
                   W0[:D], W0[D:], r2(b0), W1[:D], W1[D:], r2(b1),
                   r2(gamma0), r2(beta0), r2(gamma1), r2(beta1))


# speedup vs baseline: 7.3203x; 1.0324x over previous
import functools

import jax
import jax.numpy as jnp
from jax import lax
from jax.experimental import pallas as pl
from jax.experimental.pallas import tpu as pltpu
from jax.experimental.pallas import tpu_sc as plsc

N = 100000
E = 3200000
D = 128
DE = 16
OUT = 128

NUM_SC = 2
NUM_SUBCORES = 16
NUM_WORKERS = NUM_SC * NUM_SUBCORES

NPAD = 100096
ACC_LEN = NPAD * DE
ZSLICE = ACC_LEN // NUM_SUBCORES

TOTAL_ELEMS = E * DE
SLAB = 6400
NUM_SLABS = TOTAL_ELEMS // SLAB
SLABS_PER_W = NUM_SLABS // NUM_WORKERS

EC = E // 128


def _sc_segment_sum(vals1, oidx2, zeros1):
  mesh = plsc.VectorSubcoreMesh(core_axis_name="c", subcore_axis_name="s")

  @functools.partial(
      pl.kernel,
      out_type=jax.ShapeDtypeStruct((NUM_SC * ACC_LEN,), jnp.float32),
      mesh=mesh,
      scratch_types=[
          pltpu.VMEM((SLAB,), jnp.float32),
          pltpu.VMEM((SLAB,), jnp.float32),
          pltpu.VMEM((SLAB,), jnp.int32),
          pltpu.VMEM((SLAB,), jnp.int32),
          pltpu.VMEM_SHARED((ACC_LEN,), jnp.float32),
          pltpu.SemaphoreType.DMA,
          pltpu.SemaphoreType.DMA,
      ],
  )
  def sc_kernel(vals_hbm, idx_hbm, zeros_hbm, out_hbm, val0, val1, idx0,
                idx1, acc, sem0, sem1):
    cid = lax.axis_index("c")
    sid = lax.axis_index("s")
    wid = sid * NUM_SC + cid
    bufs = ((val0, idx0, sem0), (val1, idx1, sem1))

    def issue(b, c):
      vv, iv, sem = bufs[b]
      pltpu.async_copy(vals_hbm.at[pl.ds(c * SLAB, SLAB)], vv, sem)
      pltpu.async_copy(idx_hbm.at[pl.ds(c * SLAB, SLAB)], iv, sem)

    def wait_and_scatter(b, c):
      vv, iv, sem = bufs[b]
      pltpu.make_async_copy(vals_hbm.at[pl.ds(c * SLAB, SLAB)], vv, sem).wait()
      pltpu.make_async_copy(idx_hbm.at[pl.ds(c * SLAB, SLAB)], iv, sem).wait()
      pltpu.sync_copy(vv, acc.at[iv], add=True)

    z0 = sid * ZSLICE
    pltpu.sync_copy(zeros_hbm.at[pl.ds(z0, ZSLICE)], acc.at[pl.ds(z0, ZSLICE)])
    plsc.subcore_barrier()

    issue(0, wid)
    issue(1, wid + NUM_WORKERS)

    @pl.loop(0, SLABS_PER_W, step=2)
    def _(i):
      for b in range(2):
        c = wid + (i + b) * NUM_WORKERS
        wait_and_scatter(b, c)

        @pl.when(i + b + 2 < SLABS_PER_W)
        def _():
          issue(b, c + 2 * NUM_WORKERS)

    plsc.subcore_barrier()
    pltpu.sync_copy(acc.at[pl.ds(z0, ZSLICE)],
                    out_hbm.at[pl.ds(cid * ACC_LEN + z0, ZSLICE)])

  return sc_kernel(vals1, oidx2, zeros1)


ROW_BLOCK = 2048
PACK_ROWS = ROW_BLOCK * DE // 128


def _tc_dense_kernel(p_ref, nrep_ref, x_ref, w0h_ref, w0a_ref, b0_ref,
                     w1h_ref, w1a_ref, b1_ref, g0_ref, bt0_ref, g1_ref,
                     bt1_ref, out_ref):
  ap = (p_ref[0] + p_ref[1]) * nrep_ref[...]

  def an_dot(wa):
    parts = [
        jnp.dot(ap[:, 16 * g:16 * (g + 1)], wa,
                preferred_element_type=jnp.float32,
                precision=lax.Precision.HIGHEST)
        for g in range(8)
    ]
    t = jnp.stack(parts, axis=1)
    return t.reshape(ROW_BLOCK, OUT)

  h = x_ref[...]
  for wh, wa, b, g, bt in ((w0h_ref, w0a_ref, b0_ref, g0_ref, bt0_ref),
                           (w1h_ref, w1a_ref, b1_ref, g1_ref, bt1_ref)):
    t = (jnp.dot(h, wh[...], preferred_element_type=jnp.float32,
                 precision=lax.Precision.HIGHEST)
         + an_dot(wa[...]) + b[...])
    mean = jnp.mean(t, axis=-1, keepdims=True)
    var = jnp.mean((t - mean) ** 2, axis=-1, keepdims=True)
    t = (t - mean) / jnp.sqrt(var + 1e-5) * g[...] + bt[...]
    h = jnp.maximum(t, 0.0)
  out_ref[...] = h


def _tc_dense(partials, nrep, x, W0h, W0a, b0, W1h, W1a, b1, g0, bt0, g1,
              bt1):
  nblk = -(-N // ROW_BLOCK)
  full = lambda *shape: pl.BlockSpec(shape, lambda i: (0,) * len(shape))
  return pl.pallas_call(
      _tc_dense_kernel,
      grid=(nblk,),
      in_specs=[
          pl.BlockSpec((NUM_SC, PACK_ROWS, 128), lambda i: (0, i, 0)),
          pl.BlockSpec((PACK_ROWS, 128), lambda i: (i, 0)),
          pl.BlockSpec((ROW_BLOCK, D), lambda i: (i, 0)),
          full(D, OUT), full(DE, OUT), full(1, OUT),
          full(D, OUT), full(DE, OUT), full(1, OUT),
          full(1, OUT), full(1, OUT), full(1, OUT), full(1, OUT),
      ],
      out_specs=pl.BlockSpec((ROW_BLOCK, OUT), lambda i: (i, 0)),
      out_shape=jax.ShapeDtypeStruct((N, OUT), jnp.float32),
  )(partials, nrep, x, W0h, W0a, b0, W1h, W1a, b1, g0, bt0, g1, bt1)


@jax.jit
def kernel(x, edge_attr, norm, W0, b0, gamma0, beta0, W1, b1, gamma1, beta1,
           edge_index):
  vals1 = (edge_attr.T.reshape(2, 8, EC, 128)
           .transpose(0, 2, 1, 3).reshape(TOTAL_ELEMS))
  dst = edge_index[1].reshape(EC, 128)
  oidx = (dst[None, :, None, :] * DE
          + 8 * jnp.arange(2, dtype=jnp.int32)[:, None, None, None]
          + jnp.arange(8, dtype=jnp.int32)[None, None, :, None])
  oidx2 = oidx.reshape(TOTAL_ELEMS)
  zeros1 = jnp.zeros((ACC_LEN,), jnp.float32)

  partials = _sc_segment_sum(vals1, oidx2, zeros1)
  packed = partials.reshape(NUM_SC, ACC_LEN // 128, 128)

  nrep = jnp.repeat(norm.reshape(N), DE).reshape(N * DE // 128, 128)

  r2 = lambda v: v.reshape(1, OUT)
  return _tc_dense(packed, nrep, x,
                   W0[:D], W0[D:], r2(b0), W1[:D], W1[D:], r2(b1),
                   r2(gamma0), r2(beta0), r2(gamma1), r2(beta1))


# speedup vs baseline: 7.4734x; 1.0209x over previous
import functools

import jax
import jax.numpy as jnp
from jax import lax
from jax.experimental import pallas as pl
from jax.experimental.pallas import tpu as pltpu
from jax.experimental.pallas import tpu_sc as plsc

N = 100000
E = 3200000
D = 128
DE = 16
OUT = 128

NUM_SC = 2
NUM_SUBCORES = 16
NUM_WORKERS = NUM_SC * NUM_SUBCORES

NPAD = 100096
ACC_LEN = NPAD * DE
ZSLICE = ACC_LEN // NUM_SUBCORES

TOTAL_ELEMS = E * DE
SLAB = 6400
NUM_SLABS = TOTAL_ELEMS // SLAB
SLABS_PER_W = NUM_SLABS // NUM_WORKERS

EC = E // 128


def _sc_segment_sum(vals1, oidx2, zeros1):
  mesh = plsc.VectorSubcoreMesh(core_axis_name="c", subcore_axis_name="s")

  @functools.partial(
      pl.kernel,
      out_type=jax.ShapeDtypeStruct((NUM_SC * ACC_LEN,), jnp.float32),
      mesh=mesh,
      scratch_types=[
          pltpu.VMEM((SLAB,), jnp.float32),
          pltpu.VMEM((SLAB,), jnp.float32),
          pltpu.VMEM((SLAB,), jnp.int32),
          pltpu.VMEM((SLAB,), jnp.int32),
          pltpu.VMEM_SHARED((ACC_LEN,), jnp.float32),
          pltpu.SemaphoreType.DMA,
          pltpu.SemaphoreType.DMA,
      ],
  )
  def sc_kernel(vals_hbm, idx_hbm, zeros_hbm, out_hbm, val0, val1, idx0,
                idx1, acc, sem0, sem1):
    cid = lax.axis_index("c")
    sid = lax.axis_index("s")
    wid = sid * NUM_SC + cid
    bufs = ((val0, idx0, sem0), (val1, idx1, sem1))

    def issue(b, c):
      vv, iv, sem = bufs[b]
      pltpu.async_copy(vals_hbm.at[pl.ds(c * SLAB, SLAB)], vv, sem)
      pltpu.async_copy(idx_hbm.at[pl.ds(c * SLAB, SLAB)], iv, sem)

    def wait_and_scatter(b, c):
      vv, iv, sem = bufs[b]
      pltpu.make_async_copy(vals_hbm.at[pl.ds(c * SLAB, SLAB)], vv, sem).wait()
      pltpu.make_async_copy(idx_hbm.at[pl.ds(c * SLAB, SLAB)], iv, sem).wait()
      pltpu.sync_copy(vv, acc.at[iv], add=True)

    z0 = sid * ZSLICE
    pltpu.sync_copy(zeros_hbm.at[pl.ds(z0, ZSLICE)], acc.at[pl.ds(z0, ZSLICE)])
    plsc.subcore_barrier()

    issue(0, wid)
    issue(1, wid + NUM_WORKERS)

    @pl.loop(0, SLABS_PER_W, step=2)
    def _(i):
      for b in range(2):
        c = wid + (i + b) * NUM_WORKERS
        wait_and_scatter(b, c)

        @pl.when(i + b + 2 < SLABS_PER_W)
        def _():
          issue(b, c + 2 * NUM_WORKERS)

    plsc.subcore_barrier()
    pltpu.sync_copy(acc.at[pl.ds(z0, ZSLICE)],
                    out_hbm.at[pl.ds(cid * ACC_LEN + z0, ZSLICE)])

  return sc_kernel(vals1, oidx2, zeros1)


ROW_BLOCK = 2048
PACK_ROWS = ROW_BLOCK * DE // 128


def _tc_dense_kernel(p_ref, nrep_ref, x_ref, w0h_ref, w0a_ref, b0_ref,
                     w1h_ref, w1a_ref, b1_ref, g0_ref, bt0_ref, g1_ref,
                     bt1_ref, out_ref):
  ap = (p_ref[0] + p_ref[1]) * nrep_ref[...]

  def an_dot(wexp):
    t = jnp.dot(ap, wexp, preferred_element_type=jnp.float32,
                precision=lax.Precision.HIGHEST)
    return t.reshape(PACK_ROWS, 8, OUT).reshape(ROW_BLOCK, OUT)

  h = x_ref[...]
  for wh, wa, b, g, bt in ((w0h_ref, w0a_ref, b0_ref, g0_ref, bt0_ref),
                           (w1h_ref, w1a_ref, b1_ref, g1_ref, bt1_ref)):
    t = (jnp.dot(h, wh[...], preferred_element_type=jnp.float32,
                 precision=lax.Precision.HIGHEST)
         + an_dot(wa[...]) + b[...])
    mean = jnp.mean(t, axis=-1, keepdims=True)
    var = jnp.mean((t - mean) ** 2, axis=-1, keepdims=True)
    t = (t - mean) / jnp.sqrt(var + 1e-5) * g[...] + bt[...]
    h = jnp.maximum(t, 0.0)
  out_ref[...] = h


def _tc_dense(partials, nrep, x, W0h, W0a, b0, W1h, W1a, b1, g0, bt0, g1,
              bt1):
  nblk = -(-N // ROW_BLOCK)
  full = lambda *shape: pl.BlockSpec(shape, lambda i: (0,) * len(shape))
  return pl.pallas_call(
      _tc_dense_kernel,
      grid=(nblk,),
      in_specs=[
          pl.BlockSpec((NUM_SC, PACK_ROWS, 128), lambda i: (0, i, 0)),
          pl.BlockSpec((PACK_ROWS, 128), lambda i: (i, 0)),
          pl.BlockSpec((ROW_BLOCK, D), lambda i: (i, 0)),
          full(D, OUT), full(D, 8 * OUT), full(1, OUT),
          full(D, OUT), full(D, 8 * OUT), full(1, OUT),
          full(1, OUT), full(1, OUT), full(1, OUT), full(1, OUT),
      ],
      out_specs=pl.BlockSpec((ROW_BLOCK, OUT), lambda i: (i, 0)),
      out_shape=jax.ShapeDtypeStruct((N, OUT), jnp.float32),
  )(partials, nrep, x, W0h, W0a, b0, W1h, W1a, b1, g0, bt0, g1, bt1)


@jax.jit
def kernel(x, edge_attr, norm, W0, b0, gamma0, beta0, W1, b1, gamma1, beta1,
           edge_index):
  vals1 = (edge_attr.T.reshape(2, 8, EC, 128)
           .transpose(0, 2, 1, 3).reshape(TOTAL_ELEMS))
  dst = edge_index[1].reshape(EC, 128)
  oidx = (dst[None, :, None, :] * DE
          + 8 * jnp.arange(2, dtype=jnp.int32)[:, None, None, None]
          + jnp.arange(8, dtype=jnp.int32)[None, None, :, None])
  oidx2 = oidx.reshape(TOTAL_ELEMS)
  zeros1 = jnp.zeros((ACC_LEN,), jnp.float32)

  partials = _sc_segment_sum(vals1, oidx2, zeros1)
  packed = partials.reshape(NUM_SC, ACC_LEN // 128, 128)

  nrep = jnp.repeat(norm.reshape(N), DE).reshape(N * DE // 128, 128)

  r2 = lambda v: v.reshape(1, OUT)

  eye8 = jnp.eye(8, dtype=jnp.float32)
  wexp = lambda wa: (jnp.einsum("gh,fo->gfho", eye8, wa)
                     .reshape(D, 8 * OUT))

  return _tc_dense(packed, nrep, x,
                   W0[:D], wexp(W0[D:]), r2(b0), W1[:D], wexp(W1[D:]), r2(b1),
                   r2(gamma0), r2(beta0), r2(gamma1), r2(beta1))


# speedup vs baseline: 8.3108x; 1.1120x over previous
import functools

import jax
import jax.numpy as jnp
from jax import lax
from jax.experimental import pallas as pl
from jax.experimental.pallas import tpu as pltpu
from jax.experimental.pallas import tpu_sc as plsc

N = 100000
E = 3200000
D = 128
DE = 16
OUT = 128

NUM_SC = 2
NUM_SUBCORES = 16
NUM_WORKERS = NUM_SC * NUM_SUBCORES

NPAD = 100096
ACC_LEN = NPAD * DE
ZSLICE = ACC_LEN // NUM_SUBCORES

TOTAL_ELEMS = E * DE
SLAB = 6400
NUM_SLABS = TOTAL_ELEMS // SLAB
SLABS_PER_W = NUM_SLABS // NUM_WORKERS

EC = E // 128


def _sc_segment_sum(vals1, oidx2, zeros1):
  mesh = plsc.VectorSubcoreMesh(core_axis_name="c", subcore_axis_name="s")

  @functools.partial(
      pl.kernel,
      out_type=jax.ShapeDtypeStruct((NUM_SC * ACC_LEN,), jnp.float32),
      mesh=mesh,
      scratch_types=[
          pltpu.VMEM((SLAB,), jnp.float32),
          pltpu.VMEM((SLAB,), jnp.float32),
          pltpu.VMEM((SLAB,), jnp.int32),
          pltpu.VMEM((SLAB,), jnp.int32),
          pltpu.VMEM_SHARED((ACC_LEN,), jnp.float32),
          pltpu.SemaphoreType.DMA,
          pltpu.SemaphoreType.DMA,
      ],
  )
  def sc_kernel(vals_hbm, idx_hbm, zeros_hbm, out_hbm, val0, val1, idx0,
                idx1, acc, sem0, sem1):
    cid = lax.axis_index("c")
    sid = lax.axis_index("s")
    wid = sid * NUM_SC + cid
    bufs = ((val0, idx0, sem0), (val1, idx1, sem1))

    def issue(b, c):
      vv, iv, sem = bufs[b]
      pltpu.async_copy(vals_hbm.at[pl.ds(c * SLAB, SLAB)], vv, sem)
      pltpu.async_copy(idx_hbm.at[pl.ds(c * SLAB, SLAB)], iv, sem)

    def wait_and_scatter(b, c):
      vv, iv, sem = bufs[b]
      pltpu.make_async_copy(vals_hbm.at[pl.ds(c * SLAB, SLAB)], vv, sem).wait()
      pltpu.make_async_copy(idx_hbm.at[pl.ds(c * SLAB, SLAB)], iv, sem).wait()
      pltpu.sync_copy(vv, acc.at[iv], add=True)

    z0 = sid * ZSLICE
    pltpu.sync_copy(zeros_hbm.at[pl.ds(z0, ZSLICE)], acc.at[pl.ds(z0, ZSLICE)])
    plsc.subcore_barrier()

    issue(0, wid)
    issue(1, wid + NUM_WORKERS)

    @pl.loop(0, SLABS_PER_W, step=2)
    def _(i):
      for b in range(2):
        c = wid + (i + b) * NUM_WORKERS
        wait_and_scatter(b, c)

        @pl.when(i + b + 2 < SLABS_PER_W)
        def _():
          issue(b, c + 2 * NUM_WORKERS)

    plsc.subcore_barrier()
    pltpu.sync_copy(acc.at[pl.ds(z0, ZSLICE)],
                    out_hbm.at[pl.ds(cid * ACC_LEN + z0, ZSLICE)])

  return sc_kernel(vals1, oidx2, zeros1)


ROW_BLOCK = 2048
PACK_ROWS = ROW_BLOCK * DE // 128


def _tc_dense_kernel(p_ref, nrep_ref, x_ref, w0h_ref, w0a_ref, b0_ref,
                     w1h_ref, w1a_ref, b1_ref, g0_ref, bt0_ref, g1_ref,
                     bt1_ref, out_ref):
  ap = (p_ref[0] + p_ref[1]) * nrep_ref[...]

  def an_dot(wexp):
    t = jnp.dot(ap, wexp, preferred_element_type=jnp.float32)
    return t.reshape(PACK_ROWS, 8, OUT).reshape(ROW_BLOCK, OUT)

  h = x_ref[...]
  for wh, wa, b, g, bt in ((w0h_ref, w0a_ref, b0_ref, g0_ref, bt0_ref),
                           (w1h_ref, w1a_ref, b1_ref, g1_ref, bt1_ref)):
    t = (jnp.dot(h, wh[...], preferred_element_type=jnp.float32)
         + an_dot(wa[...]) + b[...])
    mean = jnp.mean(t, axis=-1, keepdims=True)
    var = jnp.mean((t - mean) ** 2, axis=-1, keepdims=True)
    t = (t - mean) / jnp.sqrt(var + 1e-5) * g[...] + bt[...]
    h = jnp.maximum(t, 0.0)
  out_ref[...] = h


def _tc_dense(partials, nrep, x, W0h, W0a, b0, W1h, W1a, b1, g0, bt0, g1,
              bt1):
  nblk = -(-N // ROW_BLOCK)
  full = lambda *shape: pl.BlockSpec(shape, lambda i: (0,) * len(shape))
  return pl.pallas_call(
      _tc_dense_kernel,
      grid=(nblk,),
      in_specs=[
          pl.BlockSpec((NUM_SC, PACK_ROWS, 128), lambda i: (0, i, 0)),
          pl.BlockSpec((PACK_ROWS, 128), lambda i: (i, 0)),
          pl.BlockSpec((ROW_BLOCK, D), lambda i: (i, 0)),
          full(D, OUT), full(D, 8 * OUT), full(1, OUT),
          full(D, OUT), full(D, 8 * OUT), full(1, OUT),
          full(1, OUT), full(1, OUT), full(1, OUT), full(1, OUT),
      ],
      out_specs=pl.BlockSpec((ROW_BLOCK, OUT), lambda i: (i, 0)),
      out_shape=jax.ShapeDtypeStruct((N, OUT), jnp.float32),
  )(partials, nrep, x, W0h, W0a, b0, W1h, W1a, b1, g0, bt0, g1, bt1)


@jax.jit
def kernel(x, edge_attr, norm, W0, b0, gamma0, beta0, W1, b1, gamma1, beta1,
           edge_index):
  vals1 = (edge_attr.T.reshape(2, 8, EC, 128)
           .transpose(0, 2, 1, 3).reshape(TOTAL_ELEMS))
  dst = edge_index[1].reshape(EC, 128)
  oidx = (dst[None, :, None, :] * DE
          + 8 * jnp.arange(2, dtype=jnp.int32)[:, None, None, None]
          + jnp.arange(8, dtype=jnp.int32)[None, None, :, None])
  oidx2 = oidx.reshape(TOTAL_ELEMS)
  zeros1 = jnp.zeros((ACC_LEN,), jnp.float32)

  partials = _sc_segment_sum(vals1, oidx2, zeros1)
  packed = partials.reshape(NUM_SC, ACC_LEN // 128, 128)

  nrep = jnp.repeat(norm.reshape(N), DE).reshape(N * DE // 128, 128)

  r2 = lambda v: v.reshape(1, OUT)

  eye8 = jnp.eye(8, dtype=jnp.float32)
  wexp = lambda wa: (jnp.einsum("gh,fo->gfho", eye8, wa)
                     .reshape(D, 8 * OUT))

  return _tc_dense(packed, nrep, x,
                   W0[:D], wexp(W0[D:]), r2(b0), W1[:D], wexp(W1[D:]), r2(b1),
                   r2(gamma0), r2(beta0), r2(gamma1), r2(beta1))


# speedup vs baseline: 8.3186x; 1.0009x over previous
import functools

import jax
import jax.numpy as jnp
from jax import lax
from jax.experimental import pallas as pl
from jax.experimental.pallas import tpu as pltpu
from jax.experimental.pallas import tpu_sc as plsc

N = 100000
E = 3200000
D = 128
DE = 16
OUT = 128

NUM_SC = 2
NUM_SUBCORES = 16
NUM_WORKERS = NUM_SC * NUM_SUBCORES

NPAD = 100096
ACC_LEN = NPAD * DE
ZSLICE = ACC_LEN // NUM_SUBCORES

TOTAL_ELEMS = E * DE
SLAB = 6400
NUM_SLABS = TOTAL_ELEMS // SLAB
SLABS_PER_W = NUM_SLABS // NUM_WORKERS

EC = E // 128


def _sc_segment_sum(vals1, oidx2, zeros1):
  mesh = plsc.VectorSubcoreMesh(core_axis_name="c", subcore_axis_name="s")

  @functools.partial(
      pl.kernel,
      out_type=jax.ShapeDtypeStruct((NUM_SC * ACC_LEN,), jnp.float32),
      mesh=mesh,
      scratch_types=[
          pltpu.VMEM((SLAB,), jnp.float32),
          pltpu.VMEM((SLAB,), jnp.float32),
          pltpu.VMEM((SLAB,), jnp.int32),
          pltpu.VMEM((SLAB,), jnp.int32),
          pltpu.VMEM_SHARED((ACC_LEN,), jnp.float32),
          pltpu.SemaphoreType.DMA,
          pltpu.SemaphoreType.DMA,
      ],
  )
  def sc_kernel(vals_hbm, idx_hbm, zeros_hbm, out_hbm, val0, val1, idx0,
                idx1, acc, sem0, sem1):
    cid = lax.axis_index("c")
    sid = lax.axis_index("s")
    wid = sid * NUM_SC + cid
    bufs = ((val0, idx0, sem0), (val1, idx1, sem1))

    def issue(b, c):
      vv, iv, sem = bufs[b]
      pltpu.async_copy(vals_hbm.at[pl.ds(c * SLAB, SLAB)], vv, sem)
      pltpu.async_copy(idx_hbm.at[pl.ds(c * SLAB, SLAB)], iv, sem)

    def wait_and_scatter(b, c):
      vv, iv, sem = bufs[b]
      pltpu.make_async_copy(vals_hbm.at[pl.ds(c * SLAB, SLAB)], vv, sem).wait()
      pltpu.make_async_copy(idx_hbm.at[pl.ds(c * SLAB, SLAB)], iv, sem).wait()
      pltpu.sync_copy(vv, acc.at[iv], add=True)

    z0 = sid * ZSLICE
    pltpu.sync_copy(zeros_hbm.at[pl.ds(z0, ZSLICE)], acc.at[pl.ds(z0, ZSLICE)])
    plsc.subcore_barrier()

    issue(0, wid)
    issue(1, wid + NUM_WORKERS)

    @pl.loop(0, SLABS_PER_W, step=2)
    def _(i):
      for b in range(2):
        c = wid + (i + b) * NUM_WORKERS
        wait_and_scatter(b, c)

        @pl.when(i + b + 2 < SLABS_PER_W)
        def _():
          issue(b, c + 2 * NUM_WORKERS)

    plsc.subcore_barrier()
    pltpu.sync_copy(acc.at[pl.ds(z0, ZSLICE)],
                    out_hbm.at[pl.ds(cid * ACC_LEN + z0, ZSLICE)])

  return sc_kernel(vals1, oidx2, zeros1)


ROW_BLOCK = 4096
PACK_ROWS = ROW_BLOCK * DE // 128


def _tc_dense_kernel(p_ref, nrep_ref, x_ref, w0h_ref, w0a_ref, b0_ref,
                     w1h_ref, w1a_ref, b1_ref, g0_ref, bt0_ref, g1_ref,
                     bt1_ref, out_ref):
  ap = (p_ref[0] + p_ref[1]) * nrep_ref[...]

  def an_dot(wexp):
    t = jnp.dot(ap, wexp, preferred_element_type=jnp.float32)
    return t.reshape(PACK_ROWS, 8, OUT).reshape(ROW_BLOCK, OUT)

  h = x_ref[...]
  for wh, wa, b, g, bt in ((w0h_ref, w0a_ref, b0_ref, g0_ref, bt0_ref),
                           (w1h_ref, w1a_ref, b1_ref, g1_ref, bt1_ref)):
    t = (jnp.dot(h, wh[...], preferred_element_type=jnp.float32)
         + an_dot(wa[...]) + b[...])
    mean = jnp.mean(t, axis=-1, keepdims=True)
    var = jnp.mean((t - mean) ** 2, axis=-1, keepdims=True)
    t = (t - mean) / jnp.sqrt(var + 1e-5) * g[...] + bt[...]
    h = jnp.maximum(t, 0.0)
  out_ref[...] = h


def _tc_dense(partials, nrep, x, W0h, W0a, b0, W1h, W1a, b1, g0, bt0, g1,
              bt1):
  nblk = -(-N // ROW_BLOCK)
  full = lambda *shape: pl.BlockSpec(shape, lambda i: (0,) * len(shape))
  return pl.pallas_call(
      _tc_dense_kernel,
      grid=(nblk,),
      in_specs=[
          pl.BlockSpec((NUM_SC, PACK_ROWS, 128), lambda i: (0, i, 0)),
          pl.BlockSpec((PACK_ROWS, 128), lambda i: (i, 0)),
          pl.BlockSpec((ROW_BLOCK, D), lambda i: (i, 0)),
          full(D, OUT), full(D, 8 * OUT), full(1, OUT),
          full(D, OUT), full(D, 8 * OUT), full(1, OUT),
          full(1, OUT), full(1, OUT), full(1, OUT), full(1, OUT),
      ],
      out_specs=pl.BlockSpec((ROW_BLOCK, OUT), lambda i: (i, 0)),
      out_shape=jax.ShapeDtypeStruct((N, OUT), jnp.float32),
  )(partials, nrep, x, W0h, W0a, b0, W1h, W1a, b1, g0, bt0, g1, bt1)


@jax.jit
def kernel(x, edge_attr, norm, W0, b0, gamma0, beta0, W1, b1, gamma1, beta1,
           edge_index):
  vals1 = (edge_attr.T.reshape(2, 8, EC, 128)
           .transpose(0, 2, 1, 3).reshape(TOTAL_ELEMS))
  dst = edge_index[1].reshape(EC, 128)
  oidx = (dst[None, :, None, :] * DE
          + 8 * jnp.arange(2, dtype=jnp.int32)[:, None, None, None]
          + jnp.arange(8, dtype=jnp.int32)[None, None, :, None])
  oidx2 = oidx.reshape(TOTAL_ELEMS)
  zeros1 = jnp.zeros((ACC_LEN,), jnp.float32)

  partials = _sc_segment_sum(vals1, oidx2, zeros1)
  packed = partials.reshape(NUM_SC, ACC_LEN // 128, 128)

  nrep = jnp.repeat(norm.reshape(N), DE).reshape(N * DE // 128, 128)

  r2 = lambda v: v.reshape(1, OUT)

  eye8 = jnp.eye(8, dtype=jnp.float32)
  wexp = lambda wa: (jnp.einsum("gh,fo->gfho", eye8, wa)
                     .reshape(D, 8 * OUT))

  return _tc_dense(packed, nrep, x,
                   W0[:D], wexp(W0[D:]), r2(b0), W1[:D], wexp(W1[D:]), r2(b1),
                   r2(gamma0), r2(beta0), r2(gamma1), r2(beta1))
